# Initial kernel scaffold; baseline (speedup 1.0000x reference)
import jax
import jax.numpy as jnp
from jax.experimental import pallas as pl


def kernel(ref_bxyz, ref_feat, query_bxyz, e_ref, e_query, e_weight, W_l0, g_l0, b_l0, W_f0, g_f0, b_f0, W1, b1, g1, bb1):
    raise NotImplementedError("write your pallas kernel here")


# speedup vs baseline: 3.0108x; 3.0108x over previous
import functools

import jax
import jax.numpy as jnp
from jax import lax
from jax.experimental import pallas as pl
from jax.experimental.pallas import tpu as pltpu
from jax.experimental.pallas import tpu_sc as plsc

NW = 32
LANES = 16


def _mesh():
    return plsc.VectorSubcoreMesh(core_axis_name="c", subcore_axis_name="s")


def _wid():
    return lax.axis_index("c") * 16 + lax.axis_index("s")


def _sc_dstat(ref_bxyz_flat, query_bxyz_flat, e_ref, e_query):
    E = e_ref.shape[0]
    NTAB = ref_bxyz_flat.shape[0]
    CH = 2000
    per_tile = E // NW
    nchunks = per_tile // CH
    assert per_tile % CH == 0

    def body(refb, qryb, er, eq, out, rtab, qtab, erbuf, eqbuf, accbuf,
             sem_r, sem_q):
        wid = _wid()
        tile_base = wid * per_tile
        cp_r = pltpu.async_copy(refb, rtab, sem_r)
        cp_q = pltpu.async_copy(qryb, qtab, sem_q)
        cp_r.wait()
        cp_q.wait()
        lid = lax.iota(jnp.int32, LANES)

        def chunk(i, accs):
            base = tile_base + i * CH
            pltpu.sync_copy(er.at[pl.ds(base, CH)], erbuf)
            pltpu.sync_copy(eq.at[pl.ds(base, CH)], eqbuf)

            def grp(g, accs):
                (ax, ay, az, axx, ayy, azz, axy, ayz, azx) = accs
                er4 = erbuf[pl.ds(g * LANES, LANES)] * 4
                eq4 = eqbuf[pl.ds(g * LANES, LANES)] * 4
                dx = (plsc.load_gather(rtab, [er4 + 1])
                      - plsc.load_gather(qtab, [eq4 + 1]))
                dy = (plsc.load_gather(rtab, [er4 + 2])
                      - plsc.load_gather(qtab, [eq4 + 2]))
                dz = (plsc.load_gather(rtab, [er4 + 3])
                      - plsc.load_gather(qtab, [eq4 + 3]))
                return (ax + dx, ay + dy, az + dz,
                        axx + dx * dx, ayy + dy * dy, azz + dz * dz,
                        axy + dx * dy, ayz + dy * dz, azx + dz * dx)

            return lax.fori_loop(0, CH // LANES, grp, accs)

        zero = jnp.zeros((LANES,), jnp.float32)
        accs = lax.fori_loop(0, nchunks, chunk, (zero,) * 9)
        for k in range(9):
            accbuf[k, :] = accs[k]
        pltpu.sync_copy(accbuf, out.at[wid])

    fn = pl.kernel(
        body,
        out_type=jax.ShapeDtypeStruct((NW, 9, LANES), jnp.float32),
        mesh=_mesh(),
        compiler_params=pltpu.CompilerParams(use_tc_tiling_on_sc=False, needs_layout_passes=False),
        scratch_types=[
            pltpu.VMEM((NTAB,), jnp.float32),
            pltpu.VMEM((NTAB,), jnp.float32),
            pltpu.VMEM((CH,), jnp.int32),
            pltpu.VMEM((CH,), jnp.int32),
            pltpu.VMEM((9, LANES), jnp.float32),
            pltpu.SemaphoreType.DMA,
            pltpu.SemaphoreType.DMA,
        ],
    )
    return fn(ref_bxyz_flat, query_bxyz_flat, e_ref, e_query)


def _tc_prep(dstat, ref_bxyz, query_bxyz, ref_feat,
             W_l0, g_l0, b_l0, W_f0, g_f0, b_f0, n_edges):
    N = ref_bxyz.shape[0]
    H = W_l0.shape[1]

    def body(ds_ref, rb_ref, qb_ref, rf_ref, wl_ref, gl_ref, bl_ref,
             wf_ref, gf_ref, bf_ref, u_ref, nv_ref):
        ds = ds_ref[...]
        tot = jnp.sum(ds, axis=(0, 2))
        e = jnp.float32(n_edges)
        sx, sy, sz = tot[0] / e, tot[1] / e, tot[2] / e
        sxx, syy, szz = tot[3] / e, tot[4] / e, tot[5] / e
        sxy, syz, szx = tot[6] / e, tot[7] / e, tot[8] / e

        wl = wl_ref[...]
        wx, wy, wz = wl[0, :], wl[1, :], wl[2, :]
        mean_y = sx * wx + sy * wy + sz * wz
        ey2 = (sxx * wx * wx + syy * wy * wy + szz * wz * wz
               + 2.0 * (sxy * wx * wy + syz * wy * wz + szx * wz * wx))
        var_y = ey2 - mean_y * mean_y
        a1 = gl_ref[...] / jnp.sqrt(var_y + 1e-3)
        c1 = bl_ref[...] - a1 * mean_y

        y = jnp.dot(rf_ref[...], wf_ref[...],
                    preferred_element_type=jnp.float32)
        mu = jnp.mean(y, axis=0)
        var = jnp.mean(y * y, axis=0) - mu * mu
        rf2 = gf_ref[...] * (y - mu) / jnp.sqrt(var + 1e-3) + bf_ref[...]

        a0 = jnp.dot(rb_ref[...][:, 1:4], wl,
                     preferred_element_type=jnp.float32)
        b0 = jnp.dot(qb_ref[...][:, 1:4], wl,
                     preferred_element_type=jnp.float32)
        u_ref[...] = a1 * a0 + rf2 + c1
        nv_ref[...] = -(a1 * b0)

    return pl.pallas_call(
        body,
        out_shape=[
            jax.ShapeDtypeStruct((N, H), jnp.float32),
            jax.ShapeDtypeStruct((N, H), jnp.float32),
        ],
    )(dstat, ref_bxyz, query_bxyz, ref_feat, W_l0, g_l0, b_l0,
      W_f0, g_f0, b_f0)


def _sc_gather(U, negV, e_ref, e_query):
    E = e_ref.shape[0]
    H = U.shape[1]
    CH = 128
    K = 4
    SUP = CH * K
    nsup = E // SUP
    assert E % SUP == 0
    sup_per_tile = (nsup + NW - 1) // NW

    def body(u_h, nv_h, er_h, eq_h, out_h, ir, iq, gbuf, sem_i, sem_g,
             sem_a, sem_o):
        wid = _wid()

        def sup(s, _):
            sc = wid + NW * s

            @pl.when(sc < nsup)
            def _():
                base = sc * SUP
                cps = [pltpu.async_copy(
                    er_h.at[pl.ds(base + j * CH, CH)], ir.at[j], sem_i)
                    for j in range(K)]
                cps += [pltpu.async_copy(
                    eq_h.at[pl.ds(base + j * CH, CH)], iq.at[j], sem_i)
                    for j in range(K)]
                for cp in cps:
                    cp.wait()
                cps = [pltpu.async_copy(u_h.at[ir.at[j]], gbuf.at[j], sem_g)
                       for j in range(K)]
                for cp in cps:
                    cp.wait()
                cps = [pltpu.async_copy(nv_h.at[iq.at[j]], gbuf.at[j],
                                        sem_a, add=True)
                       for j in range(K)]
                for cp in cps:
                    cp.wait()
                cps = [pltpu.async_copy(
                    gbuf.at[j], out_h.at[pl.ds(base + j * CH, CH)], sem_o)
                    for j in range(K)]
                for cp in cps:
                    cp.wait()

            return 0

        lax.fori_loop(0, sup_per_tile, sup, 0)

    fn = pl.kernel(
        body,
        out_type=jax.ShapeDtypeStruct((E, H), jnp.float32),
        mesh=_mesh(),
        compiler_params=pltpu.CompilerParams(use_tc_tiling_on_sc=False, needs_layout_passes=False),
        scratch_types=[
            pltpu.VMEM((K, CH), jnp.int32),
            pltpu.VMEM((K, CH), jnp.int32),
            pltpu.VMEM((K, CH, H), jnp.float32),
            pltpu.SemaphoreType.DMA,
            pltpu.SemaphoreType.DMA,
            pltpu.SemaphoreType.DMA,
            pltpu.SemaphoreType.DMA,
        ],
    )
    return fn(U, negV, e_ref, e_query)


def _tc_zstats(g, W1, b1_row):
    E, H = g.shape
    BE = 2560
    grid = E // BE
    assert E % BE == 0

    def body(g_ref, w_ref, b_ref, zs_ref, zq_ref):
        f = jnp.maximum(g_ref[...], 0.0)
        z = jnp.dot(f, w_ref[...], preferred_element_type=jnp.float32)
        z = z + b_ref[...]

        @pl.when(pl.program_id(0) == 0)
        def _():
            zs_ref[...] = jnp.zeros_like(zs_ref)
            zq_ref[...] = jnp.zeros_like(zq_ref)

        zs_ref[...] += jnp.sum(z, axis=0, keepdims=True)
        zq_ref[...] += jnp.sum(z * z, axis=0, keepdims=True)

    return pl.pallas_call(
        body,
        grid=(grid,),
        in_specs=[
            pl.BlockSpec((BE, H), lambda i: (i, 0)),
            pl.BlockSpec((H, H), lambda i: (0, 0)),
            pl.BlockSpec((1, H), lambda i: (0, 0)),
        ],
        out_specs=[
            pl.BlockSpec((1, H), lambda i: (0, 0)),
            pl.BlockSpec((1, H), lambda i: (0, 0)),
        ],
        out_shape=[
            jax.ShapeDtypeStruct((1, H), jnp.float32),
            jax.ShapeDtypeStruct((1, H), jnp.float32),
        ],
    )(g, W1, b1_row)


def _tc_obuild(g, W1, b1_row, g1_row, bb1_row, zsum, zsq):
    E, H = g.shape
    BE = 2560
    grid = E // BE

    def body(g_ref, w_ref, b_ref, g1_ref, bb_ref, zs_ref, zq_ref, o_ref):
        e = jnp.float32(E)
        mu = zs_ref[...] / e
        var = zq_ref[...] / e - mu * mu
        a2 = g1_ref[...] * lax.rsqrt(var + 1e-3)
        c2 = bb_ref[...] - a2 * mu
        f = jnp.maximum(g_ref[...], 0.0)
        z = jnp.dot(f, w_ref[...], preferred_element_type=jnp.float32)
        z = z + b_ref[...]
        o = jnp.maximum(a2 * z + c2, 0.0)
        o_ref[...] = o.T

    return pl.pallas_call(
        body,
        grid=(grid,),
        in_specs=[
            pl.BlockSpec((BE, H), lambda i: (i, 0)),
            pl.BlockSpec((H, H), lambda i: (0, 0)),
            pl.BlockSpec((1, H), lambda i: (0, 0)),
            pl.BlockSpec((1, H), lambda i: (0, 0)),
            pl.BlockSpec((1, H), lambda i: (0, 0)),
            pl.BlockSpec((1, H), lambda i: (0, 0)),
            pl.BlockSpec((1, H), lambda i: (0, 0)),
        ],
        out_specs=pl.BlockSpec((H, BE), lambda i: (0, i)),
        out_shape=jax.ShapeDtypeStruct((H, E), jnp.float32),
    )(g, W1, b1_row, g1_row, bb1_row, zsum, zsq)


def _sc_segmax(oT, e_query, N):
    H, E = oT.shape
    NCG = 8
    NEQ = NW // NCG
    CPG = H // NCG
    EQ_E = E // NEQ
    CH = 800
    nchunks = EQ_E // CH
    assert EQ_E % CH == 0 and CH % LANES == 0

    def body(ot_h, eq_h, out_h, qbuf, obuf, acc, tmp, scr16, shiftbuf,
             sem_q, sem_o):
        wid = _wid()
        cg = wid % NCG
        eq = wid // NCG
        c0 = cg * CPG
        lid = lax.iota(jnp.int32, LANES)
        zero16 = jnp.zeros((LANES,), jnp.float32)
        shiftbuf[pl.ds(0, LANES)] = jnp.full((LANES,), -1, jnp.int32)

        def zrow(i, _):
            acc[pl.ds(i * LANES, LANES)] = zero16
            return 0
        lax.fori_loop(0, (CPG * N) // LANES, zrow, 0)

        def chunk(k, _):
            base = eq * EQ_E + k * CH
            cpq = pltpu.async_copy(eq_h.at[pl.ds(base, CH)], qbuf, sem_q)
            cps = [pltpu.async_copy(
                ot_h.at[c0 + c, pl.ds(base, CH)], obuf.at[c], sem_o)
                for c in range(CPG)]
            cpq.wait()
            for cp in cps:
                cp.wait()

            def grp(gi, _):
                qv = qbuf[pl.ds(gi * LANES, LANES)]
                plsc.store_scatter(tmp, [qv], lid)
                back = plsc.load_gather(tmp, [qv])
                has_dup = lax.reduce_max(
                    (back != lid).astype(jnp.int32), (0,)) > 0

                def upd(c, mask):
                    vals = obuf[c, pl.ds(gi * LANES, LANES)]
                    aflat = acc
                    idx = qv + c * N
                    cur = plsc.load_gather(aflat, [idx], mask=mask)
                    plsc.store_scatter(aflat, [idx],
                                       jnp.maximum(cur, vals), mask=mask)

                def fast():
                    ones = lid >= 0
                    for c in range(CPG):
                        upd(c, ones)

                def slow():
                    sk, sv = plsc.sort_key_val(qv, lid)
                    shiftbuf[pl.ds(1, LANES)] = sk
                    prev = shiftbuf[pl.ds(0, LANES)]
                    isfirst = (lid == 0) | (sk != prev)
                    segstart = plsc.cummax(jnp.where(isfirst, lid, 0))
                    occ_sorted = lid - segstart
                    plsc.store_scatter(scr16, [sv], occ_sorted)
                    occ = scr16[...]
                    rounds = lax.reduce_max(occ_sorted, (0,)) + 1

                    def rbody(r, _):
                        m = occ == r
                        for c in range(CPG):
                            upd(c, m)
                        return 0

                    lax.fori_loop(0, rounds, rbody, 0)

                lax.cond(has_dup, slow, fast)
                return 0

            lax.fori_loop(0, CH // LANES, grp, 0)
            return 0

        lax.fori_loop(0, nchunks, chunk, 0)
        pltpu.sync_copy(acc, out_h.at[eq, pl.ds(c0 * N, CPG * N)])

    fn = pl.kernel(
        body,
        out_type=jax.ShapeDtypeStruct((NEQ, H * N), jnp.float32),
        mesh=_mesh(),
        compiler_params=pltpu.CompilerParams(use_tc_tiling_on_sc=False, needs_layout_passes=False),
        scratch_types=[
            pltpu.VMEM((CH,), jnp.int32),
            pltpu.VMEM((CPG, CH), jnp.float32),
            pltpu.VMEM((CPG * N,), jnp.float32),
            pltpu.VMEM((N,), jnp.int32),
            pltpu.VMEM((LANES,), jnp.int32),
            pltpu.VMEM((2 * LANES,), jnp.int32),
            pltpu.SemaphoreType.DMA,
            pltpu.SemaphoreType.DMA,
        ],
    )
    return fn(oT, e_query).reshape(NEQ, H, N)


def _tc_finish(P):
    NEQ, H, N = P.shape

    def body(p_ref, o_ref):
        m = jnp.max(p_ref[...], axis=0)
        o_ref[...] = m.T

    return pl.pallas_call(
        body,
        out_shape=jax.ShapeDtypeStruct((N, H), jnp.float32),
    )(P)


def kernel(ref_bxyz, ref_feat, query_bxyz, e_ref, e_query, e_weight,
           W_l0, g_l0, b_l0, W_f0, g_f0, b_f0, W1, b1, g1, bb1):
    del e_weight
    N = ref_bxyz.shape[0]
    E = e_ref.shape[0]
    H = W_l0.shape[1]

    dstat = _sc_dstat(ref_bxyz.reshape(-1), query_bxyz.reshape(-1),
                      e_ref, e_query)
    U, negV = _tc_prep(dstat, ref_bxyz, query_bxyz, ref_feat,
                       W_l0, g_l0, b_l0, W_f0, g_f0, b_f0, E)
    g = _sc_gather(U, negV, e_ref, e_query)
    b1_row = b1.reshape(1, H)
    zsum, zsq = _tc_zstats(g, W1, b1_row)
    oT = _tc_obuild(g, W1, b1_row, g1.reshape(1, H), bb1.reshape(1, H),
                    zsum, zsq)
    P = _sc_segmax(oT, e_query, N)
    return _tc_finish(P)
